# baseline (speedup 1.0000x reference)
import functools

import jax
import jax.numpy as jnp
from jax import lax
from jax.experimental import pallas as pl
from jax.experimental.pallas import tpu as pltpu
from jax.experimental.pallas import tpu_sc as plsc

_N = 10000
_G = 128
_D = 128
_L = 16
_NT = 16
_CHUNK = 624
_HALF = 144
_ACC = 2 * _HALF
_NB = _ACC // _L
_MAXG = 40

_mesh = plsc.VectorSubcoreMesh(
    core_axis_name="c", subcore_axis_name="s", num_cores=1)


@functools.partial(
    pl.kernel,
    out_type=(
        jax.ShapeDtypeStruct((_G,), jnp.float32),
        jax.ShapeDtypeStruct((_G,), jnp.float32),
    ),
    mesh=_mesh,
    compiler_params=pltpu.CompilerParams(
        needs_layout_passes=False, skip_device_barrier=True,
        use_tc_tiling_on_sc=False),
    scratch_types=[
        pltpu.VMEM((_MAXG * _L, _L), jnp.float32),
        pltpu.VMEM((_MAXG * _L,), jnp.int32),
        pltpu.VMEM((_ACC,), jnp.float32),
        pltpu.VMEM((2, _NT, _L), jnp.float32),
        pltpu.VMEM((2 * _L,), jnp.float32),
        pltpu.VMEM_SHARED((_NT, _ACC), jnp.float32),
        pltpu.SemaphoreType.DMA,
    ],
)
def _seg_mean(x_hbm, batch_hbm, mean_out, std_out,
              rows_v, bat_v, acc_v, red_v, out_v, shr, sem):
  wid = lax.axis_index("s")
  base = wid * _CHUNK
  iota = lax.iota(jnp.int32, _L)
  zeros_f = jnp.zeros((_L,), jnp.float32)
  zeros_i = jnp.zeros((_L,), jnp.int32)
  ones_f = zeros_f + jnp.float32(1.0)

  d_bat = pltpu.async_copy(batch_hbm.at[pl.ds(base, _CHUNK)],
                           bat_v.at[pl.ds(0, _CHUNK)], sem)
  _RC = _MAXG * _L // 4
  d_rows = [
      pltpu.async_copy(x_hbm.at[pl.ds(base + k * _RC, _RC), pl.ds(0, _L)],
                       rows_v.at[pl.ds(k * _RC, _RC)], sem)
      for k in range(4)
  ]

  @pl.when(wid == _NT - 1)
  def _():
    pltpu.sync_copy(batch_hbm.at[pl.ds(_NT * _CHUNK, _L)],
                    bat_v.at[pl.ds(_CHUNK, _L)])

  for j in range(_NB):
    acc_v[pl.ds(j * _L, _L)] = zeros_f
  d_bat.wait()

  def group(g):
    b0 = g * _L
    s = bat_v[pl.ds(b0, _L)]
    v = plsc.load_gather(rows_v, [b0 + iota, zeros_i])
    plsc.addupdate_scatter(acc_v, [s], v)
    plsc.addupdate_scatter(acc_v, [s + _HALF], ones_f)

  for k in range(4):
    d_rows[k].wait()
    lo = k * _RC // _L
    hi = min((k + 1) * _RC // _L, _CHUNK // _L)
    plsc.parallel_loop(lo, hi, unroll=5)(group)

  @pl.when(wid == _NT - 1)
  def _():
    plsc.parallel_loop(_CHUNK // _L, _MAXG)(group)

  pltpu.sync_copy(acc_v, shr.at[wid])
  plsc.subcore_barrier()

  @pl.when(wid < _G // _L)
  def _():
    d1 = pltpu.async_copy(shr.at[pl.ds(0, _NT), pl.ds(wid * _L, _L)],
                          red_v.at[0], sem)
    d2 = pltpu.async_copy(shr.at[pl.ds(0, _NT), pl.ds(_HALF + wid * _L, _L)],
                          red_v.at[1], sem)
    d1.wait()
    d2.wait()
    tot = red_v[0, 0]
    cnt = red_v[1, 0]
    for t in range(1, _NT):
      tot = tot + red_v[0, t]
      cnt = cnt + red_v[1, t]
    out_v[pl.ds(0, _L)] = tot / cnt
    pltpu.sync_copy(out_v.at[pl.ds(0, _L)],
                    mean_out.at[pl.ds(wid * _L, _L)])

  @pl.when(wid >= _G // _L)
  def _():
    out_v[pl.ds(_L, _L)] = zeros_f + jnp.float32(0.1)
    pltpu.sync_copy(out_v.at[pl.ds(_L, _L)],
                    std_out.at[pl.ds((wid - _G // _L) * _L, _L)])


def kernel(x, edge_index, edge_attr, batch):
  del edge_index, edge_attr
  mean, std = _seg_mean(x, batch)
  return mean.reshape(_G, 1), std.reshape(_G, 1)


# speedup vs baseline: 1.0065x; 1.0065x over previous
import functools

import jax
import jax.numpy as jnp
from jax import lax
from jax.experimental import pallas as pl
from jax.experimental.pallas import tpu as pltpu
from jax.experimental.pallas import tpu_sc as plsc

_N = 10000
_G = 128
_D = 128
_L = 16
_NT = 16
_CHUNK = 624
_HALF = 144
_ACC = 2 * _HALF
_NB = _ACC // _L
_MAXG = 40

_mesh = plsc.VectorSubcoreMesh(
    core_axis_name="c", subcore_axis_name="s", num_cores=1)


@functools.partial(
    pl.kernel,
    out_type=(
        jax.ShapeDtypeStruct((_G,), jnp.float32),
        jax.ShapeDtypeStruct((_G,), jnp.float32),
    ),
    mesh=_mesh,
    compiler_params=pltpu.CompilerParams(
        needs_layout_passes=False, skip_device_barrier=True,
        use_tc_tiling_on_sc=False),
    scratch_types=[
        pltpu.VMEM((_MAXG * _L, _L), jnp.float32),
        pltpu.VMEM((_MAXG * _L,), jnp.int32),
        pltpu.VMEM((_ACC,), jnp.float32),
        pltpu.VMEM((2, _NT, _L), jnp.float32),
        pltpu.VMEM((2 * _L,), jnp.float32),
        pltpu.VMEM_SHARED((_NT, _ACC), jnp.float32),
        pltpu.SemaphoreType.DMA,
    ],
)
def _seg_mean(x_hbm, batch_hbm, mean_out, std_out,
              rows_v, bat_v, acc_v, red_v, out_v, shr, sem):
  wid = lax.axis_index("s")
  base = wid * _CHUNK
  iota = lax.iota(jnp.int32, _L)
  zeros_f = jnp.zeros((_L,), jnp.float32)
  zeros_i = jnp.zeros((_L,), jnp.int32)
  ones_f = zeros_f + jnp.float32(1.0)

  d_bat = pltpu.async_copy(batch_hbm.at[pl.ds(base, _MAXG * _L)],
                           bat_v.at[pl.ds(0, _MAXG * _L)], sem)
  _RC = _MAXG * _L // 4
  d_rows = [
      pltpu.async_copy(x_hbm.at[pl.ds(base + k * _RC, _RC), pl.ds(0, _L)],
                       rows_v.at[pl.ds(k * _RC, _RC)], sem)
      for k in range(4)
  ]

  for j in range(_NB):
    acc_v[pl.ds(j * _L, _L)] = zeros_f
  d_bat.wait()

  def group(g):
    b0 = g * _L
    s = bat_v[pl.ds(b0, _L)]
    v = plsc.load_gather(rows_v, [b0 + iota, zeros_i])
    plsc.addupdate_scatter(acc_v, [s], v)
    plsc.addupdate_scatter(acc_v, [s + _HALF], ones_f)

  for k in range(4):
    d_rows[k].wait()
    lo = k * _RC // _L
    hi = min((k + 1) * _RC // _L, _CHUNK // _L)
    plsc.parallel_loop(lo, hi, unroll=2)(group)

  @pl.when(wid == _NT - 1)
  def _():
    plsc.parallel_loop(_CHUNK // _L, _MAXG)(group)

  pltpu.sync_copy(acc_v, shr.at[wid])
  plsc.subcore_barrier()

  @pl.when(wid < _G // _L)
  def _():
    d1 = pltpu.async_copy(shr.at[pl.ds(0, _NT), pl.ds(wid * _L, _L)],
                          red_v.at[0], sem)
    d2 = pltpu.async_copy(shr.at[pl.ds(0, _NT), pl.ds(_HALF + wid * _L, _L)],
                          red_v.at[1], sem)
    d1.wait()
    d2.wait()
    tot = red_v[0, 0]
    cnt = red_v[1, 0]
    for t in range(1, _NT):
      tot = tot + red_v[0, t]
      cnt = cnt + red_v[1, t]
    out_v[pl.ds(0, _L)] = tot / cnt
    pltpu.sync_copy(out_v.at[pl.ds(0, _L)],
                    mean_out.at[pl.ds(wid * _L, _L)])

  @pl.when(wid >= _G // _L)
  def _():
    out_v[pl.ds(_L, _L)] = zeros_f + jnp.float32(0.1)
    pltpu.sync_copy(out_v.at[pl.ds(_L, _L)],
                    std_out.at[pl.ds((wid - _G // _L) * _L, _L)])


def kernel(x, edge_index, edge_attr, batch):
  del edge_index, edge_attr
  mean, std = _seg_mean(x, batch)
  return mean.reshape(_G, 1), std.reshape(_G, 1)
